# baseline (speedup 1.0000x reference)
import functools

import jax
import jax.numpy as jnp
from jax import lax
from jax.experimental import pallas as pl
from jax.experimental.pallas import tpu as pltpu
from jax.experimental.pallas import tpu_sc as plsc

N = 10000
E = 160000
HID = 256
HEADS = 8
HD = 32
NPAD = 10240
CW = 128
NCHUNK = E // CW
NB = 1000
EBK = 2000
F32 = jnp.float32


def _f32(x):
    return x.astype(jnp.float32)


def _ln_block(x, g, b, eps=1e-5):
    m = jnp.mean(x, axis=-1, keepdims=True)
    v = jnp.mean((x - m) ** 2, axis=-1, keepdims=True)
    return (x - m) * jax.lax.rsqrt(v + eps) * g + b


def _dot(a, b):
    return jnp.dot(a, b, preferred_element_type=jnp.float32)


def _tc_in(x, w, b):
    def body(x_ref, w_ref, b_ref, o_ref):
        o_ref[...] = _dot(x_ref[...], w_ref[...]) + b_ref[...]

    return pl.pallas_call(
        body,
        grid=(N // NB,),
        in_specs=[
            pl.BlockSpec((NB, HID), lambda i: (i, 0)),
            pl.BlockSpec((HID, HID), lambda i: (0, 0)),
            pl.BlockSpec((1, HID), lambda i: (0, 0)),
        ],
        out_specs=pl.BlockSpec((NB, HID), lambda i: (i, 0)),
        out_shape=jax.ShapeDtypeStruct((N, HID), F32),
    )(x, w, b)


def _tc_eb(curv, c1w, c1b, c2w, wb, c2b, bbp):
    def body(c_ref, c1w_ref, c1b_ref, c2w_ref, wb_ref, c2b_ref, bbp_ref,
             o_ref, w4_ref, k_ref):
        @pl.when(pl.program_id(0) == 0)
        def _():
            w4_ref[...] = _dot(c2w_ref[...], wb_ref[...])
            k_ref[...] = _dot(c2b_ref[...], wb_ref[...]) + bbp_ref[...]

        r = jnp.maximum(c_ref[...] * c1w_ref[...] + c1b_ref[...], 0.0)
        o_ref[...] = _dot(r, w4_ref[...]) + k_ref[...]

    return pl.pallas_call(
        body,
        grid=(E // EBK,),
        in_specs=[
            pl.BlockSpec((EBK, 1), lambda i: (i, 0)),
            pl.BlockSpec((1, HID), lambda i: (0, 0)),
            pl.BlockSpec((1, HID), lambda i: (0, 0)),
            pl.BlockSpec((HID, HID), lambda i: (0, 0)),
            pl.BlockSpec((HID, 64), lambda i: (0, 0)),
            pl.BlockSpec((1, HID), lambda i: (0, 0)),
            pl.BlockSpec((1, 64), lambda i: (0, 0)),
        ],
        out_specs=pl.BlockSpec((EBK, 64), lambda i: (i, 0)),
        out_shape=jax.ShapeDtypeStruct((E, 64), F32),
        scratch_shapes=[
            pltpu.VMEM((HID, 64), F32),
            pltpu.VMEM((1, 64), F32),
        ],
    )(curv, c1w, c1b, c2w, wb, c2b, bbp)


def _tc_qkv(h, g, bln, wq, bq, wk, bk, wv, bv):
    def body(h_ref, g_ref, b_ref, wq_ref, bq_ref, wk_ref, bk_ref,
             wv_ref, bv_ref, q_ref, k_ref, v_ref):
        hn = _ln_block(h_ref[...], g_ref[...], b_ref[...])
        q_ref[...] = _dot(hn, wq_ref[...]) + bq_ref[...]
        k_ref[...] = _dot(hn, wk_ref[...]) + bk_ref[...]
        v_ref[...] = _dot(hn, wv_ref[...]) + bv_ref[...]

    wspec = pl.BlockSpec((HID, HID), lambda i: (0, 0))
    bspec = pl.BlockSpec((1, HID), lambda i: (0, 0))
    nspec = pl.BlockSpec((NB, HID), lambda i: (i, 0))
    sds = jax.ShapeDtypeStruct((N, HID), F32)
    return pl.pallas_call(
        body,
        grid=(N // NB,),
        in_specs=[nspec, bspec, bspec, wspec, bspec, wspec, bspec, wspec, bspec],
        out_specs=[nspec, nspec, nspec],
        out_shape=[sds, sds, sds],
    )(h, g, bln, wq, bq, wk, bk, wv, bv)


def _tc_scores(qs, kt, eb_all, layer):
    def body(qs_ref, kt_ref, eb_ref, o_ref, ow_ref):
        d = lax.broadcasted_iota(jnp.int32, (HID, 16), 0)
        hh = lax.broadcasted_iota(jnp.int32, (HID, 16), 1)
        m = jnp.where(d // HD == hh, 1.0 / jnp.sqrt(jnp.float32(HD)), 0.0)
        s = _dot(qs_ref[...] * kt_ref[...], m)
        eb = eb_ref[:, 16 * layer:16 * layer + 16]
        ex = jnp.exp(s + eb)
        o_ref[...] = ex
        ow_ref[...] = jnp.concatenate(
            [ex, jnp.zeros((EBK, 112), F32)], axis=1)

    return pl.pallas_call(
        body,
        grid=(E // EBK,),
        in_specs=[
            pl.BlockSpec((EBK, HID), lambda i: (i, 0)),
            pl.BlockSpec((EBK, HID), lambda i: (i, 0)),
            pl.BlockSpec((EBK, 64), lambda i: (i, 0)),
        ],
        out_specs=[pl.BlockSpec((EBK, 16), lambda i: (i, 0)),
                   pl.BlockSpec((EBK, 128), lambda i: (i, 0))],
        out_shape=[jax.ShapeDtypeStruct((E, 16), F32),
                   jax.ShapeDtypeStruct((E, 128), F32)],
    )(qs, kt, eb_all)


def _tc_recip(den):
    def body(d_ref, o_ref):
        s = d_ref[0, :, :16] + d_ref[1, :, :16]
        o_ref[...] = 1.0 / (s + 1e-16)

    return pl.pallas_call(
        body,
        grid=(NPAD // NB,),
        in_specs=[pl.BlockSpec((2, NB, 128), lambda i: (0, i, 0))],
        out_specs=pl.BlockSpec((NB, 16), lambda i: (i, 0)),
        out_shape=jax.ShapeDtypeStruct((NPAD, 16), F32),
    )(den)


def _tc_vr(v, recip):
    def body(v_ref, r_ref, o_ref):
        hh = lax.broadcasted_iota(jnp.int32, (16, HID), 0)
        d = lax.broadcasted_iota(jnp.int32, (16, HID), 1)
        s = jnp.where(d // HD == hh, 1.0, 0.0)
        o_ref[...] = v_ref[...] * _dot(r_ref[...], s)

    return pl.pallas_call(
        body,
        grid=(N // NB,),
        in_specs=[
            pl.BlockSpec((NB, HID), lambda i: (i, 0)),
            pl.BlockSpec((NB, 16), lambda i: (i, 0)),
        ],
        out_specs=pl.BlockSpec((NB, HID), lambda i: (i, 0)),
        out_shape=jax.ShapeDtypeStruct((N, HID), F32),
    )(v, recip)


def _tc_msgs(ex, vrt):
    def body(ex_ref, vt_ref, o_ref):
        hh = lax.broadcasted_iota(jnp.int32, (16, HID), 0)
        d = lax.broadcasted_iota(jnp.int32, (16, HID), 1)
        s = jnp.where(d // HD == hh, 1.0, 0.0)
        m = _dot(ex_ref[...], s) * vt_ref[...]
        o_ref[0] = m[:, :128]
        o_ref[1] = m[:, 128:]

    return pl.pallas_call(
        body,
        grid=(E // EBK,),
        in_specs=[
            pl.BlockSpec((EBK, 16), lambda i: (i, 0)),
            pl.BlockSpec((EBK, HID), lambda i: (i, 0)),
        ],
        out_specs=pl.BlockSpec((2, EBK, 128), lambda i: (0, i, 0)),
        out_shape=jax.ShapeDtypeStruct((2, E, 128), F32),
    )(ex, vrt)


def _tc_post(h, agg, wo, bo, g2, b2ln, w1, b1, w2, b2):
    def body(h_ref, a_ref, wo_ref, bo_ref, g_ref, bl_ref, w1_ref, b1_ref,
             w2_ref, b2_ref, o_ref):
        a = jnp.concatenate([a_ref[0], a_ref[1]], axis=1)
        h1 = h_ref[...] + _dot(a, wo_ref[...]) + bo_ref[...]
        hn2 = _ln_block(h1, g_ref[...], bl_ref[...])
        f = jnp.maximum(_dot(hn2, w1_ref[...]) + b1_ref[...], 0.0)
        o_ref[...] = h1 + _dot(f, w2_ref[...]) + b2_ref[...]

    bspec = pl.BlockSpec((1, HID), lambda i: (0, 0))
    return pl.pallas_call(
        body,
        grid=(N // NB,),
        in_specs=[
            pl.BlockSpec((NB, HID), lambda i: (i, 0)),
            pl.BlockSpec((2, NB, 128), lambda i: (0, i, 0)),
            pl.BlockSpec((HID, HID), lambda i: (0, 0)),
            bspec, bspec, bspec,
            pl.BlockSpec((HID, 4 * HID), lambda i: (0, 0)),
            pl.BlockSpec((1, 4 * HID), lambda i: (0, 0)),
            pl.BlockSpec((4 * HID, HID), lambda i: (0, 0)),
            bspec,
        ],
        out_specs=pl.BlockSpec((NB, HID), lambda i: (i, 0)),
        out_shape=jax.ShapeDtypeStruct((N, HID), F32),
    )(h, agg, wo, bo, g2, b2ln, w1, b1, w2, b2)


def _tc_out(h, w, b):
    def body(h_ref, w_ref, b_ref, o_ref):
        o_ref[...] = _dot(h_ref[...], w_ref[...]) + b_ref[...]

    return pl.pallas_call(
        body,
        grid=(N // NB,),
        in_specs=[
            pl.BlockSpec((NB, HID), lambda i: (i, 0)),
            pl.BlockSpec((HID, 128), lambda i: (0, 0)),
            pl.BlockSpec((1, 128), lambda i: (0, 0)),
        ],
        out_specs=pl.BlockSpec((NB, 128), lambda i: (i, 0)),
        out_shape=jax.ShapeDtypeStruct((N, 128), F32),
    )(h, w, b)


@functools.lru_cache(maxsize=1)
def _sc_mesh():
    return plsc.VectorSubcoreMesh(core_axis_name="c", subcore_axis_name="s")


def _stage_idx_flat(i_hbm, ibuf, start, n, cw, is_long):
    @pl.when(is_long)
    def _():
        pltpu.sync_copy(i_hbm.at[pl.ds(start * cw, (n + 1) * cw)], ibuf)

    @pl.when(jnp.logical_not(is_long))
    def _():
        pltpu.sync_copy(i_hbm.at[pl.ds(start * cw, n * cw)],
                        ibuf.at[pl.ds(0, n * cw)])


def _sc_gather_qk(q, k, src, tgt):
    n, r = NCHUNK // 16, NCHUNK % 16
    ngt = (n + 3) // 3

    @functools.partial(
        pl.kernel, mesh=_sc_mesh(),
        out_type=[jax.ShapeDtypeStruct((E, HID), F32),
                  jax.ShapeDtypeStruct((E, HID), F32)],
        scratch_types=[pltpu.VMEM(((n + 1) * CW,), jnp.int32),
                       pltpu.VMEM((CW, HID), F32), pltpu.VMEM((CW, HID), F32),
                       pltpu.VMEM((CW, HID), F32)]
                      + [pltpu.SemaphoreType.DMA] * 6,
    )
    def kfn(q_hbm, k_hbm, s_hbm, t_hbm, qs_hbm, kt_hbm,
            ibuf, b0, b1, b2, sg0, sg1, sg2, sw0, sw1, sw2):
        cid = lax.axis_index("c")
        sid = lax.axis_index("s")
        ncw = n + jnp.where(sid < r, 1, 0)
        start = sid * n + jnp.minimum(sid, r)

        bufs = (b0, b1, b2)
        sg, sw = (sg0, sg1, sg2), (sw0, sw1, sw2)

        def one_table(i_hbm, tbl_hbm, out_hbm):
            _stage_idx_flat(i_hbm, ibuf, start, n, CW, sid < r)

            def g_start(j, u):
                pltpu.make_async_copy(tbl_hbm.at[ibuf.at[pl.ds(j * CW, CW)]],
                                      bufs[u], sg[u]).start()

            def g_wait(u):
                pltpu.make_async_copy(tbl_hbm.at[ibuf.at[pl.ds(0, CW)]],
                                      bufs[u], sg[u]).wait()

            def wb_start(j, u):
                base = (start + j) * CW
                pltpu.make_async_copy(bufs[u], out_hbm.at[pl.ds(base, CW)],
                                      sw[u]).start()

            def wb_wait(u):
                pltpu.make_async_copy(bufs[u], out_hbm.at[pl.ds(0, CW)],
                                      sw[u]).wait()

            g_start(0, 0)
            g_start(1, 1)

            @pl.loop(0, ngt)
            def _(g):
                for u in (0, 1, 2):
                    j = g * 3 + u

                    @pl.when(j < ncw)
                    def _():
                        g_wait(u)
                        wb_start(j, u)

                        @pl.when(j + 2 < ncw)
                        def _():
                            @pl.when(j >= 1)
                            def _():
                                wb_wait((u + 2) % 3)

                            g_start(j + 2, (u + 2) % 3)

            wb_wait(0)
            wb_wait(1)
            wb_wait(2)

        @pl.when(cid == 0)
        def _():
            one_table(s_hbm, q_hbm, qs_hbm)

        @pl.when(cid == 1)
        def _():
            one_table(t_hbm, k_hbm, kt_hbm)

    return kfn(q, k, src, tgt)


def _sc_den(ex_wide, tgt, zeros128):
    @functools.partial(
        pl.kernel, mesh=_sc_mesh(),
        out_type=jax.ShapeDtypeStruct((2, NPAD, 128), F32),
        scratch_types=[pltpu.VMEM((CW,), jnp.int32),
                       pltpu.VMEM((CW,), jnp.int32),
                       pltpu.VMEM((CW, 128), F32),
                       pltpu.VMEM((CW, 128), F32),
                       pltpu.VMEM_SHARED((NPAD, 128), F32)]
                      + [pltpu.SemaphoreType.DMA] * 6,
    )
    def kfn(ex_hbm, t_hbm, z_hbm, den_hbm, ti0, ti1, exb0, exb1, dsh,
            sc0, sc1, si0, si1, ss0, ss1):
        cid = lax.axis_index("c")
        sid = lax.axis_index("s")
        w = sid * 2 + cid
        n, r = NCHUNK // 32, NCHUNK % 32
        ngp = (n + 2) // 2
        ncw = n + jnp.where(w < r, 1, 0)
        start = w * n + jnp.minimum(w, r)
        pltpu.sync_copy(z_hbm.at[pl.ds(sid * 640, 640)],
                        dsh.at[pl.ds(sid * 640, 640)])
        plsc.subcore_barrier()

        exb, sc = (exb0, exb1), (sc0, sc1)
        tib, si = (ti0, ti1), (si0, si1)
        ss = (ss0, ss1)

        def sc_start(u):
            pltpu.async_copy(exb[u], dsh.at[tib[u]], ss[u], add=True)

        def sc_wait(u):
            pltpu.make_async_copy(exb[u], dsh.at[tib[u]], ss[u]).wait()

        def cp_start(j, u):
            base = (start + j) * CW
            pltpu.make_async_copy(t_hbm.at[pl.ds(base, CW)], tib[u],
                                  si[u]).start()
            pltpu.make_async_copy(ex_hbm.at[pl.ds(base, CW)], exb[u],
                                  sc[u]).start()

        def cp_wait(u):
            pltpu.make_async_copy(t_hbm.at[pl.ds(0, CW)], tib[u],
                                  si[u]).wait()
            pltpu.make_async_copy(ex_hbm.at[pl.ds(0, CW)], exb[u],
                                  sc[u]).wait()

        cp_start(0, 0)

        @pl.loop(0, ngp)
        def _(g):
            for u in (0, 1):
                j = g * 2 + u

                @pl.when(j < ncw)
                def _():
                    cp_wait(u)

                    @pl.when(j + 1 < ncw)
                    def _():
                        @pl.when(j >= 1)
                        def _():
                            sc_wait(1 - u)

                        cp_start(j + 1, 1 - u)

                    sc_start(u)

        sc_wait(0)
        sc_wait(1)
        plsc.subcore_barrier()
        pltpu.sync_copy(dsh.at[pl.ds(sid * 640, 640)],
                        den_hbm.at[cid].at[pl.ds(sid * 640, 640)])

    return kfn(ex_wide, tgt, zeros128)


def _sc_gather_v(vr, tgt):
    n, r = NCHUNK // 32, NCHUNK % 32
    ngt = (n + 3) // 3

    @functools.partial(
        pl.kernel, mesh=_sc_mesh(),
        out_type=jax.ShapeDtypeStruct((E, HID), F32),
        scratch_types=[pltpu.VMEM(((n + 1) * CW,), jnp.int32),
                       pltpu.VMEM((CW, HID), F32), pltpu.VMEM((CW, HID), F32),
                       pltpu.VMEM((CW, HID), F32)]
                      + [pltpu.SemaphoreType.DMA] * 6,
    )
    def kfn(v_hbm, t_hbm, vt_hbm, tibuf, vb0, vb1, vb2,
            sg0, sg1, sg2, sw0, sw1, sw2):
        w = lax.axis_index("s") * 2 + lax.axis_index("c")
        ncw = n + jnp.where(w < r, 1, 0)
        start = w * n + jnp.minimum(w, r)
        _stage_idx_flat(t_hbm, tibuf, start, n, CW, w < r)

        vb, sg, sw = (vb0, vb1, vb2), (sg0, sg1, sg2), (sw0, sw1, sw2)

        def g_start(j, u):
            pltpu.make_async_copy(v_hbm.at[tibuf.at[pl.ds(j * CW, CW)]],
                                  vb[u], sg[u]).start()

        def g_wait(u):
            pltpu.make_async_copy(v_hbm.at[tibuf.at[pl.ds(0, CW)]],
                                  vb[u], sg[u]).wait()

        def wb_start(j, u):
            base = (start + j) * CW
            pltpu.make_async_copy(vb[u], vt_hbm.at[pl.ds(base, CW)],
                                  sw[u]).start()

        def wb_wait(u):
            pltpu.make_async_copy(vb[u], vt_hbm.at[pl.ds(0, CW)],
                                  sw[u]).wait()

        g_start(0, 0)
        g_start(1, 1)

        @pl.loop(0, ngt)
        def _(g):
            for u in (0, 1, 2):
                j = g * 3 + u

                @pl.when(j < ncw)
                def _():
                    g_wait(u)
                    wb_start(j, u)

                    @pl.when(j + 2 < ncw)
                    def _():
                        @pl.when(j >= 1)
                        def _():
                            wb_wait((u + 2) % 3)

                        g_start(j + 2, (u + 2) % 3)

        wb_wait(0)
        wb_wait(1)
        wb_wait(2)

    return kfn(vr, tgt)


def _sc_agg(msgs, src, zeros128):
    n, r = NCHUNK // 16, NCHUNK % 16
    ngp = (n + 2) // 2

    @functools.partial(
        pl.kernel, mesh=_sc_mesh(),
        out_type=jax.ShapeDtypeStruct((2, NPAD, 128), F32),
        scratch_types=[pltpu.VMEM((CW,), jnp.int32),
                       pltpu.VMEM((CW,), jnp.int32),
                       pltpu.VMEM((CW, 128), F32),
                       pltpu.VMEM((CW, 128), F32),
                       pltpu.VMEM_SHARED((NPAD, 128), F32)]
                      + [pltpu.SemaphoreType.DMA] * 6,
    )
    def kfn(m_hbm, s_hbm, z_hbm, agg_hbm, si0b, si1b, mb0, mb1, ash,
            sc0, sc1, si0, si1, ss0, ss1):
        cid = lax.axis_index("c")
        sid = lax.axis_index("s")
        ncw = n + jnp.where(sid < r, 1, 0)
        start = sid * n + jnp.minimum(sid, r)
        pltpu.sync_copy(z_hbm.at[pl.ds(sid * 640, 640)],
                        ash.at[pl.ds(sid * 640, 640)])
        plsc.subcore_barrier()

        mb, sc = (mb0, mb1), (sc0, sc1)
        sib, si = (si0b, si1b), (si0, si1)
        ss = (ss0, ss1)

        def sc_start(u):
            pltpu.async_copy(mb[u], ash.at[sib[u]], ss[u], add=True)

        def sc_wait(u):
            pltpu.make_async_copy(mb[u], ash.at[sib[u]], ss[u]).wait()

        def cp_start(j, u):
            base = (start + j) * CW
            pltpu.make_async_copy(s_hbm.at[pl.ds(base, CW)], sib[u],
                                  si[u]).start()
            pltpu.make_async_copy(m_hbm.at[cid].at[pl.ds(base, CW)], mb[u],
                                  sc[u]).start()

        def cp_wait(u):
            pltpu.make_async_copy(s_hbm.at[pl.ds(0, CW)], sib[u],
                                  si[u]).wait()
            pltpu.make_async_copy(m_hbm.at[cid].at[pl.ds(0, CW)], mb[u],
                                  sc[u]).wait()

        cp_start(0, 0)

        @pl.loop(0, ngp)
        def _(g):
            for u in (0, 1):
                j = g * 2 + u

                @pl.when(j < ncw)
                def _():
                    cp_wait(u)

                    @pl.when(j + 1 < ncw)
                    def _():
                        @pl.when(j >= 1)
                        def _():
                            sc_wait(1 - u)

                        cp_start(j + 1, 1 - u)

                    sc_start(u)

        sc_wait(0)
        sc_wait(1)
        plsc.subcore_barrier()
        pltpu.sync_copy(ash.at[pl.ds(sid * 640, 640)],
                        agg_hbm.at[cid].at[pl.ds(sid * 640, 640)])

    return kfn(msgs, src, zeros128)


def kernel(x, edge_index, curvature, params):
    p = params
    src = edge_index[0]
    tgt = edge_index[1]

    wb = jnp.concatenate(
        [jnp.pad(lp['wbias'], ((0, 0), (0, 8))) for lp in p['layers']], axis=1)
    bbp = jnp.concatenate(
        [jnp.concatenate([lp['bbias'], jnp.full((8,), -1e30, F32)])
         for lp in p['layers']])[None, :]
    c1w = p['c1w'].reshape(1, HID)
    c1b = p['c1b'][None, :]
    c2b = p['c2b'][None, :]
    out_w = jnp.pad(p['out_w'], ((0, 0), (0, 127)))
    out_b = jnp.pad(p['out_b'], (0, 127))[None, :]
    zeros128 = jnp.zeros((NPAD, 128), F32)

    eb_all = _tc_eb(curvature, c1w, c1b, p['c2w'], wb, c2b, bbp)
    h = _tc_in(x, p['in_w'], p['in_b'][None, :])

    for l, lp in enumerate(p['layers']):
        q, k, v = _tc_qkv(h, lp['ln1_g'][None, :], lp['ln1_b'][None, :],
                          lp['wq'], lp['bq'][None, :],
                          lp['wk'], lp['bk'][None, :],
                          lp['wv'], lp['bv'][None, :])
        qs, kt = _sc_gather_qk(q, k, src, tgt)
        ex, ex_wide = _tc_scores(qs, kt, eb_all, l)
        den = _sc_den(ex_wide, tgt, zeros128)
        recip = _tc_recip(den)
        vr = _tc_vr(v, recip)
        vrt = _sc_gather_v(vr, tgt)
        msgs = _tc_msgs(ex, vrt)
        agg = _sc_agg(msgs, src, zeros128)
        h = _tc_post(h, agg, lp['wo'], lp['bo'][None, :],
                     lp['ln2_g'][None, :], lp['ln2_b'][None, :],
                     lp['w1'], lp['b1'][None, :],
                     lp['w2'], lp['b2'][None, :])

    y = _tc_out(h, out_w, out_b)
    return y[:, :1]


# speedup vs baseline: 1.1862x; 1.1862x over previous
import functools

import jax
import jax.numpy as jnp
from jax import lax
from jax.experimental import pallas as pl
from jax.experimental.pallas import tpu as pltpu
from jax.experimental.pallas import tpu_sc as plsc

N = 10000
E = 160000
HID = 256
HEADS = 8
HD = 32
NPAD = 10240
CW = 128
NCHUNK = E // CW
NB = 1000
EBK = 2000
F32 = jnp.float32


def _f32(x):
    return x.astype(jnp.float32)


def _ln_block(x, g, b, eps=1e-5):
    m = jnp.mean(x, axis=-1, keepdims=True)
    v = jnp.mean((x - m) ** 2, axis=-1, keepdims=True)
    return (x - m) * jax.lax.rsqrt(v + eps) * g + b


def _dot(a, b):
    return jnp.dot(a, b, preferred_element_type=jnp.float32)


def _tc_in(x, w, b):
    def body(x_ref, w_ref, b_ref, o_ref):
        o_ref[...] = _dot(x_ref[...], w_ref[...]) + b_ref[...]

    return pl.pallas_call(
        body,
        grid=(N // NB,),
        in_specs=[
            pl.BlockSpec((NB, HID), lambda i: (i, 0)),
            pl.BlockSpec((HID, HID), lambda i: (0, 0)),
            pl.BlockSpec((1, HID), lambda i: (0, 0)),
        ],
        out_specs=pl.BlockSpec((NB, HID), lambda i: (i, 0)),
        out_shape=jax.ShapeDtypeStruct((N, HID), F32),
    )(x, w, b)


def _tc_eb(curv, c1w, c1b, c2w, wb, c2b, bbp):
    def body(c_ref, c1w_ref, c1b_ref, c2w_ref, wb_ref, c2b_ref, bbp_ref,
             o_ref, w4_ref, k_ref):
        @pl.when(pl.program_id(0) == 0)
        def _():
            w4_ref[...] = _dot(c2w_ref[...], wb_ref[...])
            k_ref[...] = _dot(c2b_ref[...], wb_ref[...]) + bbp_ref[...]

        r = jnp.maximum(c_ref[...] * c1w_ref[...] + c1b_ref[...], 0.0)
        o_ref[...] = _dot(r, w4_ref[...]) + k_ref[...]

    return pl.pallas_call(
        body,
        grid=(E // EBK,),
        in_specs=[
            pl.BlockSpec((EBK, 1), lambda i: (i, 0)),
            pl.BlockSpec((1, HID), lambda i: (0, 0)),
            pl.BlockSpec((1, HID), lambda i: (0, 0)),
            pl.BlockSpec((HID, HID), lambda i: (0, 0)),
            pl.BlockSpec((HID, 64), lambda i: (0, 0)),
            pl.BlockSpec((1, HID), lambda i: (0, 0)),
            pl.BlockSpec((1, 64), lambda i: (0, 0)),
        ],
        out_specs=pl.BlockSpec((EBK, 64), lambda i: (i, 0)),
        out_shape=jax.ShapeDtypeStruct((E, 64), F32),
        scratch_shapes=[
            pltpu.VMEM((HID, 64), F32),
            pltpu.VMEM((1, 64), F32),
        ],
    )(curv, c1w, c1b, c2w, wb, c2b, bbp)


def _pack_bf16(x, rows):
    xi = lax.bitcast_convert_type(x, jnp.uint32)
    hi = (xi + jnp.uint32(0x8000)) >> jnp.uint32(16)
    packed = hi[:, :128] | (hi[:, 128:] << jnp.uint32(16))
    return lax.bitcast_convert_type(packed, F32)


def _unpack_bf16(xp, rows):
    p = lax.bitcast_convert_type(xp, jnp.uint32)
    a = lax.bitcast_convert_type(p << jnp.uint32(16), F32)
    b = lax.bitcast_convert_type(p & jnp.uint32(0xFFFF0000), F32)
    return jnp.concatenate([a, b], axis=1)


def _tc_qkv(h, g, bln, wq, bq, wk, bk, wv, bv):
    def body(h_ref, g_ref, b_ref, wq_ref, bq_ref, wk_ref, bk_ref,
             wv_ref, bv_ref, q_ref, k_ref, v_ref):
        hn = _ln_block(h_ref[...], g_ref[...], b_ref[...])
        q_ref[...] = _pack_bf16(_dot(hn, wq_ref[...]) + bq_ref[...], NB)
        k_ref[...] = _pack_bf16(_dot(hn, wk_ref[...]) + bk_ref[...], NB)
        v_ref[...] = _dot(hn, wv_ref[...]) + bv_ref[...]

    wspec = pl.BlockSpec((HID, HID), lambda i: (0, 0))
    bspec = pl.BlockSpec((1, HID), lambda i: (0, 0))
    nspec = pl.BlockSpec((NB, HID), lambda i: (i, 0))
    pspec = pl.BlockSpec((NB, 128), lambda i: (i, 0))
    sds = jax.ShapeDtypeStruct((N, HID), F32)
    pds = jax.ShapeDtypeStruct((N, 128), F32)
    return pl.pallas_call(
        body,
        grid=(N // NB,),
        in_specs=[nspec, bspec, bspec, wspec, bspec, wspec, bspec, wspec, bspec],
        out_specs=[pspec, pspec, nspec],
        out_shape=[pds, pds, sds],
    )(h, g, bln, wq, bq, wk, bk, wv, bv)


def _tc_scores(qs, kt, eb_all, layer):
    def body(qs_ref, kt_ref, eb_ref, o_ref, ow_ref):
        d = lax.broadcasted_iota(jnp.int32, (HID, 16), 0)
        hh = lax.broadcasted_iota(jnp.int32, (HID, 16), 1)
        m = jnp.where(d // HD == hh, 1.0 / jnp.sqrt(jnp.float32(HD)), 0.0)
        qs = _unpack_bf16(qs_ref[...], EBK)
        kt = _unpack_bf16(kt_ref[...], EBK)
        s = _dot(qs * kt, m)
        eb = eb_ref[:, 16 * layer:16 * layer + 16]
        ex = jnp.exp(s + eb)
        o_ref[...] = ex
        ow_ref[...] = jnp.concatenate(
            [ex, jnp.zeros((EBK, 112), F32)], axis=1)

    return pl.pallas_call(
        body,
        grid=(E // EBK,),
        in_specs=[
            pl.BlockSpec((EBK, 128), lambda i: (i, 0)),
            pl.BlockSpec((EBK, 128), lambda i: (i, 0)),
            pl.BlockSpec((EBK, 64), lambda i: (i, 0)),
        ],
        out_specs=[pl.BlockSpec((EBK, 16), lambda i: (i, 0)),
                   pl.BlockSpec((EBK, 128), lambda i: (i, 0))],
        out_shape=[jax.ShapeDtypeStruct((E, 16), F32),
                   jax.ShapeDtypeStruct((E, 128), F32)],
    )(qs, kt, eb_all)


def _tc_recip(den):
    def body(d_ref, o_ref):
        s = d_ref[0, :, :16] + d_ref[1, :, :16]
        o_ref[...] = 1.0 / (s + 1e-16)

    return pl.pallas_call(
        body,
        grid=(NPAD // NB,),
        in_specs=[pl.BlockSpec((2, NB, 128), lambda i: (0, i, 0))],
        out_specs=pl.BlockSpec((NB, 16), lambda i: (i, 0)),
        out_shape=jax.ShapeDtypeStruct((NPAD, 16), F32),
    )(den)


def _tc_vr(v, recip):
    def body(v_ref, r_ref, o_ref):
        hh = lax.broadcasted_iota(jnp.int32, (16, HID), 0)
        d = lax.broadcasted_iota(jnp.int32, (16, HID), 1)
        s = jnp.where(d // HD == hh, 1.0, 0.0)
        o_ref[...] = v_ref[...] * _dot(r_ref[...], s)

    return pl.pallas_call(
        body,
        grid=(N // NB,),
        in_specs=[
            pl.BlockSpec((NB, HID), lambda i: (i, 0)),
            pl.BlockSpec((NB, 16), lambda i: (i, 0)),
        ],
        out_specs=pl.BlockSpec((NB, HID), lambda i: (i, 0)),
        out_shape=jax.ShapeDtypeStruct((N, HID), F32),
    )(v, recip)


def _tc_msgs(ex, vrt):
    def body(ex_ref, vt_ref, o_ref):
        hh = lax.broadcasted_iota(jnp.int32, (16, HID), 0)
        d = lax.broadcasted_iota(jnp.int32, (16, HID), 1)
        s = jnp.where(d // HD == hh, 1.0, 0.0)
        m = _dot(ex_ref[...], s) * vt_ref[...]
        o_ref[0] = m[:, :128]
        o_ref[1] = m[:, 128:]

    return pl.pallas_call(
        body,
        grid=(E // EBK,),
        in_specs=[
            pl.BlockSpec((EBK, 16), lambda i: (i, 0)),
            pl.BlockSpec((EBK, HID), lambda i: (i, 0)),
        ],
        out_specs=pl.BlockSpec((2, EBK, 128), lambda i: (0, i, 0)),
        out_shape=jax.ShapeDtypeStruct((2, E, 128), F32),
    )(ex, vrt)


def _tc_post(h, agg, wo, bo, g2, b2ln, w1, b1, w2, b2):
    def body(h_ref, a_ref, wo_ref, bo_ref, g_ref, bl_ref, w1_ref, b1_ref,
             w2_ref, b2_ref, o_ref):
        a = jnp.concatenate([a_ref[0], a_ref[1]], axis=1)
        h1 = h_ref[...] + _dot(a, wo_ref[...]) + bo_ref[...]
        hn2 = _ln_block(h1, g_ref[...], bl_ref[...])
        f = jnp.maximum(_dot(hn2, w1_ref[...]) + b1_ref[...], 0.0)
        o_ref[...] = h1 + _dot(f, w2_ref[...]) + b2_ref[...]

    bspec = pl.BlockSpec((1, HID), lambda i: (0, 0))
    return pl.pallas_call(
        body,
        grid=(N // NB,),
        in_specs=[
            pl.BlockSpec((NB, HID), lambda i: (i, 0)),
            pl.BlockSpec((2, NB, 128), lambda i: (0, i, 0)),
            pl.BlockSpec((HID, HID), lambda i: (0, 0)),
            bspec, bspec, bspec,
            pl.BlockSpec((HID, 4 * HID), lambda i: (0, 0)),
            pl.BlockSpec((1, 4 * HID), lambda i: (0, 0)),
            pl.BlockSpec((4 * HID, HID), lambda i: (0, 0)),
            bspec,
        ],
        out_specs=pl.BlockSpec((NB, HID), lambda i: (i, 0)),
        out_shape=jax.ShapeDtypeStruct((N, HID), F32),
    )(h, agg, wo, bo, g2, b2ln, w1, b1, w2, b2)


def _tc_out(h, w, b):
    def body(h_ref, w_ref, b_ref, o_ref):
        o_ref[...] = _dot(h_ref[...], w_ref[...]) + b_ref[...]

    return pl.pallas_call(
        body,
        grid=(N // NB,),
        in_specs=[
            pl.BlockSpec((NB, HID), lambda i: (i, 0)),
            pl.BlockSpec((HID, 128), lambda i: (0, 0)),
            pl.BlockSpec((1, 128), lambda i: (0, 0)),
        ],
        out_specs=pl.BlockSpec((NB, 128), lambda i: (i, 0)),
        out_shape=jax.ShapeDtypeStruct((N, 128), F32),
    )(h, w, b)


@functools.lru_cache(maxsize=1)
def _sc_mesh():
    return plsc.VectorSubcoreMesh(core_axis_name="c", subcore_axis_name="s")


def _stage_idx_flat(i_hbm, ibuf, start, n, cw, is_long):
    @pl.when(is_long)
    def _():
        pltpu.sync_copy(i_hbm.at[pl.ds(start * cw, (n + 1) * cw)], ibuf)

    @pl.when(jnp.logical_not(is_long))
    def _():
        pltpu.sync_copy(i_hbm.at[pl.ds(start * cw, n * cw)],
                        ibuf.at[pl.ds(0, n * cw)])


def _sc_gather_qk(q, k, src, tgt):
    n, r = NCHUNK // 16, NCHUNK % 16
    ngt = (n + 3) // 3

    @functools.partial(
        pl.kernel, mesh=_sc_mesh(),
        out_type=[jax.ShapeDtypeStruct((E, 128), F32),
                  jax.ShapeDtypeStruct((E, 128), F32)],
        scratch_types=[pltpu.VMEM(((n + 1) * CW,), jnp.int32),
                       pltpu.VMEM((CW, 128), F32), pltpu.VMEM((CW, 128), F32),
                       pltpu.VMEM((CW, 128), F32)]
                      + [pltpu.SemaphoreType.DMA] * 6,
    )
    def kfn(q_hbm, k_hbm, s_hbm, t_hbm, qs_hbm, kt_hbm,
            ibuf, b0, b1, b2, sg0, sg1, sg2, sw0, sw1, sw2):
        cid = lax.axis_index("c")
        sid = lax.axis_index("s")
        ncw = n + jnp.where(sid < r, 1, 0)
        start = sid * n + jnp.minimum(sid, r)

        bufs = (b0, b1, b2)
        sg, sw = (sg0, sg1, sg2), (sw0, sw1, sw2)

        def one_table(i_hbm, tbl_hbm, out_hbm):
            _stage_idx_flat(i_hbm, ibuf, start, n, CW, sid < r)

            def g_start(j, u):
                pltpu.make_async_copy(tbl_hbm.at[ibuf.at[pl.ds(j * CW, CW)]],
                                      bufs[u], sg[u]).start()

            def g_wait(u):
                pltpu.make_async_copy(tbl_hbm.at[ibuf.at[pl.ds(0, CW)]],
                                      bufs[u], sg[u]).wait()

            def wb_start(j, u):
                base = (start + j) * CW
                pltpu.make_async_copy(bufs[u], out_hbm.at[pl.ds(base, CW)],
                                      sw[u]).start()

            def wb_wait(u):
                pltpu.make_async_copy(bufs[u], out_hbm.at[pl.ds(0, CW)],
                                      sw[u]).wait()

            g_start(0, 0)
            g_start(1, 1)

            @pl.loop(0, ngt)
            def _(g):
                for u in (0, 1, 2):
                    j = g * 3 + u

                    @pl.when(j < ncw)
                    def _():
                        g_wait(u)
                        wb_start(j, u)

                        @pl.when(j + 2 < ncw)
                        def _():
                            @pl.when(j >= 1)
                            def _():
                                wb_wait((u + 2) % 3)

                            g_start(j + 2, (u + 2) % 3)

            wb_wait(0)
            wb_wait(1)
            wb_wait(2)

        @pl.when(cid == 0)
        def _():
            one_table(s_hbm, q_hbm, qs_hbm)

        @pl.when(cid == 1)
        def _():
            one_table(t_hbm, k_hbm, kt_hbm)

    return kfn(q, k, src, tgt)


def _sc_den(ex_wide, tgt, zeros128):
    @functools.partial(
        pl.kernel, mesh=_sc_mesh(),
        out_type=jax.ShapeDtypeStruct((2, NPAD, 128), F32),
        scratch_types=[pltpu.VMEM((CW,), jnp.int32),
                       pltpu.VMEM((CW,), jnp.int32),
                       pltpu.VMEM((CW, 128), F32),
                       pltpu.VMEM((CW, 128), F32),
                       pltpu.VMEM_SHARED((NPAD, 128), F32)]
                      + [pltpu.SemaphoreType.DMA] * 6,
    )
    def kfn(ex_hbm, t_hbm, z_hbm, den_hbm, ti0, ti1, exb0, exb1, dsh,
            sc0, sc1, si0, si1, ss0, ss1):
        cid = lax.axis_index("c")
        sid = lax.axis_index("s")
        w = sid * 2 + cid
        n, r = NCHUNK // 32, NCHUNK % 32
        ngp = (n + 2) // 2
        ncw = n + jnp.where(w < r, 1, 0)
        start = w * n + jnp.minimum(w, r)
        pltpu.sync_copy(z_hbm.at[pl.ds(sid * 640, 640)],
                        dsh.at[pl.ds(sid * 640, 640)])
        plsc.subcore_barrier()

        exb, sc = (exb0, exb1), (sc0, sc1)
        tib, si = (ti0, ti1), (si0, si1)
        ss = (ss0, ss1)

        def sc_start(u):
            pltpu.async_copy(exb[u], dsh.at[tib[u]], ss[u], add=True)

        def sc_wait(u):
            pltpu.make_async_copy(exb[u], dsh.at[tib[u]], ss[u]).wait()

        def cp_start(j, u):
            base = (start + j) * CW
            pltpu.make_async_copy(t_hbm.at[pl.ds(base, CW)], tib[u],
                                  si[u]).start()
            pltpu.make_async_copy(ex_hbm.at[pl.ds(base, CW)], exb[u],
                                  sc[u]).start()

        def cp_wait(u):
            pltpu.make_async_copy(t_hbm.at[pl.ds(0, CW)], tib[u],
                                  si[u]).wait()
            pltpu.make_async_copy(ex_hbm.at[pl.ds(0, CW)], exb[u],
                                  sc[u]).wait()

        cp_start(0, 0)

        @pl.loop(0, ngp)
        def _(g):
            for u in (0, 1):
                j = g * 2 + u

                @pl.when(j < ncw)
                def _():
                    cp_wait(u)

                    @pl.when(j + 1 < ncw)
                    def _():
                        @pl.when(j >= 1)
                        def _():
                            sc_wait(1 - u)

                        cp_start(j + 1, 1 - u)

                    sc_start(u)

        sc_wait(0)
        sc_wait(1)
        plsc.subcore_barrier()
        pltpu.sync_copy(dsh.at[pl.ds(sid * 640, 640)],
                        den_hbm.at[cid].at[pl.ds(sid * 640, 640)])

    return kfn(ex_wide, tgt, zeros128)


def _sc_gather_v(vr, tgt):
    n, r = NCHUNK // 32, NCHUNK % 32
    ngt = (n + 3) // 3

    @functools.partial(
        pl.kernel, mesh=_sc_mesh(),
        out_type=jax.ShapeDtypeStruct((E, HID), F32),
        scratch_types=[pltpu.VMEM(((n + 1) * CW,), jnp.int32),
                       pltpu.VMEM((CW, HID), F32), pltpu.VMEM((CW, HID), F32),
                       pltpu.VMEM((CW, HID), F32)]
                      + [pltpu.SemaphoreType.DMA] * 6,
    )
    def kfn(v_hbm, t_hbm, vt_hbm, tibuf, vb0, vb1, vb2,
            sg0, sg1, sg2, sw0, sw1, sw2):
        w = lax.axis_index("s") * 2 + lax.axis_index("c")
        ncw = n + jnp.where(w < r, 1, 0)
        start = w * n + jnp.minimum(w, r)
        _stage_idx_flat(t_hbm, tibuf, start, n, CW, w < r)

        vb, sg, sw = (vb0, vb1, vb2), (sg0, sg1, sg2), (sw0, sw1, sw2)

        def g_start(j, u):
            pltpu.make_async_copy(v_hbm.at[tibuf.at[pl.ds(j * CW, CW)]],
                                  vb[u], sg[u]).start()

        def g_wait(u):
            pltpu.make_async_copy(v_hbm.at[tibuf.at[pl.ds(0, CW)]],
                                  vb[u], sg[u]).wait()

        def wb_start(j, u):
            base = (start + j) * CW
            pltpu.make_async_copy(vb[u], vt_hbm.at[pl.ds(base, CW)],
                                  sw[u]).start()

        def wb_wait(u):
            pltpu.make_async_copy(vb[u], vt_hbm.at[pl.ds(0, CW)],
                                  sw[u]).wait()

        g_start(0, 0)
        g_start(1, 1)

        @pl.loop(0, ngt)
        def _(g):
            for u in (0, 1, 2):
                j = g * 3 + u

                @pl.when(j < ncw)
                def _():
                    g_wait(u)
                    wb_start(j, u)

                    @pl.when(j + 2 < ncw)
                    def _():
                        @pl.when(j >= 1)
                        def _():
                            wb_wait((u + 2) % 3)

                        g_start(j + 2, (u + 2) % 3)

        wb_wait(0)
        wb_wait(1)
        wb_wait(2)

    return kfn(vr, tgt)


def _sc_agg(msgs, src, zeros128):
    n, r = NCHUNK // 16, NCHUNK % 16
    ngp = (n + 2) // 2

    @functools.partial(
        pl.kernel, mesh=_sc_mesh(),
        out_type=jax.ShapeDtypeStruct((2, NPAD, 128), F32),
        scratch_types=[pltpu.VMEM((CW,), jnp.int32),
                       pltpu.VMEM((CW,), jnp.int32),
                       pltpu.VMEM((CW, 128), F32),
                       pltpu.VMEM((CW, 128), F32),
                       pltpu.VMEM_SHARED((NPAD, 128), F32)]
                      + [pltpu.SemaphoreType.DMA] * 6,
    )
    def kfn(m_hbm, s_hbm, z_hbm, agg_hbm, si0b, si1b, mb0, mb1, ash,
            sc0, sc1, si0, si1, ss0, ss1):
        cid = lax.axis_index("c")
        sid = lax.axis_index("s")
        ncw = n + jnp.where(sid < r, 1, 0)
        start = sid * n + jnp.minimum(sid, r)
        pltpu.sync_copy(z_hbm.at[pl.ds(sid * 640, 640)],
                        ash.at[pl.ds(sid * 640, 640)])
        plsc.subcore_barrier()

        mb, sc = (mb0, mb1), (sc0, sc1)
        sib, si = (si0b, si1b), (si0, si1)
        ss = (ss0, ss1)

        def sc_start(u):
            pltpu.async_copy(mb[u], ash.at[sib[u]], ss[u], add=True)

        def sc_wait(u):
            pltpu.make_async_copy(mb[u], ash.at[sib[u]], ss[u]).wait()

        def cp_start(j, u):
            base = (start + j) * CW
            pltpu.make_async_copy(s_hbm.at[pl.ds(base, CW)], sib[u],
                                  si[u]).start()
            pltpu.make_async_copy(m_hbm.at[cid].at[pl.ds(base, CW)], mb[u],
                                  sc[u]).start()

        def cp_wait(u):
            pltpu.make_async_copy(s_hbm.at[pl.ds(0, CW)], sib[u],
                                  si[u]).wait()
            pltpu.make_async_copy(m_hbm.at[cid].at[pl.ds(0, CW)], mb[u],
                                  sc[u]).wait()

        cp_start(0, 0)

        @pl.loop(0, ngp)
        def _(g):
            for u in (0, 1):
                j = g * 2 + u

                @pl.when(j < ncw)
                def _():
                    cp_wait(u)

                    @pl.when(j + 1 < ncw)
                    def _():
                        @pl.when(j >= 1)
                        def _():
                            sc_wait(1 - u)

                        cp_start(j + 1, 1 - u)

                    sc_start(u)

        sc_wait(0)
        sc_wait(1)
        plsc.subcore_barrier()
        pltpu.sync_copy(ash.at[pl.ds(sid * 640, 640)],
                        agg_hbm.at[cid].at[pl.ds(sid * 640, 640)])

    return kfn(msgs, src, zeros128)


def kernel(x, edge_index, curvature, params):
    p = params
    src = edge_index[0]
    tgt = edge_index[1]

    wb = jnp.concatenate(
        [jnp.pad(lp['wbias'], ((0, 0), (0, 8))) for lp in p['layers']], axis=1)
    bbp = jnp.concatenate(
        [jnp.concatenate([lp['bbias'], jnp.full((8,), -1e30, F32)])
         for lp in p['layers']])[None, :]
    c1w = p['c1w'].reshape(1, HID)
    c1b = p['c1b'][None, :]
    c2b = p['c2b'][None, :]
    out_w = jnp.pad(p['out_w'], ((0, 0), (0, 127)))
    out_b = jnp.pad(p['out_b'], (0, 127))[None, :]
    zeros128 = jnp.zeros((NPAD, 128), F32)

    eb_all = _tc_eb(curvature, c1w, c1b, p['c2w'], wb, c2b, bbp)
    h = _tc_in(x, p['in_w'], p['in_b'][None, :])

    for l, lp in enumerate(p['layers']):
        q, k, v = _tc_qkv(h, lp['ln1_g'][None, :], lp['ln1_b'][None, :],
                          lp['wq'], lp['bq'][None, :],
                          lp['wk'], lp['bk'][None, :],
                          lp['wv'], lp['bv'][None, :])
        qs, kt = _sc_gather_qk(q, k, src, tgt)
        ex, ex_wide = _tc_scores(qs, kt, eb_all, l)
        den = _sc_den(ex_wide, tgt, zeros128)
        recip = _tc_recip(den)
        vr = _tc_vr(v, recip)
        vrt = _sc_gather_v(vr, tgt)
        msgs = _tc_msgs(ex, vrt)
        agg = _sc_agg(msgs, src, zeros128)
        h = _tc_post(h, agg, lp['wo'], lp['bo'][None, :],
                     lp['ln2_g'][None, :], lp['ln2_b'][None, :],
                     lp['w1'], lp['b1'][None, :],
                     lp['w2'], lp['b2'][None, :])

    y = _tc_out(h, out_w, out_b)
    return y[:, :1]


# speedup vs baseline: 1.3048x; 1.1000x over previous
import functools

import jax
import jax.numpy as jnp
from jax import lax
from jax.experimental import pallas as pl
from jax.experimental.pallas import tpu as pltpu
from jax.experimental.pallas import tpu_sc as plsc

N = 10000
E = 160000
HID = 256
HEADS = 8
HD = 32
NPAD = 10240
CW = 128
NCHUNK = E // CW
NB = 1000
EBK = 2000
F32 = jnp.float32


def _f32(x):
    return x.astype(jnp.float32)


def _ln_block(x, g, b, eps=1e-5):
    m = jnp.mean(x, axis=-1, keepdims=True)
    v = jnp.mean((x - m) ** 2, axis=-1, keepdims=True)
    return (x - m) * jax.lax.rsqrt(v + eps) * g + b


def _dot(a, b):
    return jnp.dot(a, b, preferred_element_type=jnp.float32)


def _tc_in(x, w, b):
    def body(x_ref, w_ref, b_ref, o_ref):
        o_ref[...] = _dot(x_ref[...], w_ref[...]) + b_ref[...]

    return pl.pallas_call(
        body,
        grid=(N // NB,),
        in_specs=[
            pl.BlockSpec((NB, HID), lambda i: (i, 0)),
            pl.BlockSpec((HID, HID), lambda i: (0, 0)),
            pl.BlockSpec((1, HID), lambda i: (0, 0)),
        ],
        out_specs=pl.BlockSpec((NB, HID), lambda i: (i, 0)),
        out_shape=jax.ShapeDtypeStruct((N, HID), F32),
    )(x, w, b)


def _tc_eb(curv, c1w, c1b, c2w, wb, c2b, bbp):
    def body(c_ref, c1w_ref, c1b_ref, c2w_ref, wb_ref, c2b_ref, bbp_ref,
             o_ref, w4_ref, k_ref):
        @pl.when(pl.program_id(0) == 0)
        def _():
            w4_ref[...] = _dot(c2w_ref[...], wb_ref[...])
            k_ref[...] = _dot(c2b_ref[...], wb_ref[...]) + bbp_ref[...]

        r = jnp.maximum(c_ref[...] * c1w_ref[...] + c1b_ref[...], 0.0)
        o_ref[...] = _dot(r, w4_ref[...]) + k_ref[...]

    return pl.pallas_call(
        body,
        grid=(E // EBK,),
        in_specs=[
            pl.BlockSpec((EBK, 1), lambda i: (i, 0)),
            pl.BlockSpec((1, HID), lambda i: (0, 0)),
            pl.BlockSpec((1, HID), lambda i: (0, 0)),
            pl.BlockSpec((HID, HID), lambda i: (0, 0)),
            pl.BlockSpec((HID, 64), lambda i: (0, 0)),
            pl.BlockSpec((1, HID), lambda i: (0, 0)),
            pl.BlockSpec((1, 64), lambda i: (0, 0)),
        ],
        out_specs=pl.BlockSpec((EBK, 64), lambda i: (i, 0)),
        out_shape=jax.ShapeDtypeStruct((E, 64), F32),
        scratch_shapes=[
            pltpu.VMEM((HID, 64), F32),
            pltpu.VMEM((1, 64), F32),
        ],
    )(curv, c1w, c1b, c2w, wb, c2b, bbp)


def _pack_bf16(x, rows):
    xi = lax.bitcast_convert_type(x, jnp.uint32)
    hi = (xi + jnp.uint32(0x8000)) >> jnp.uint32(16)
    packed = hi[:, :128] | (hi[:, 128:] << jnp.uint32(16))
    return lax.bitcast_convert_type(packed, F32)


def _unpack_bf16(xp, rows):
    p = lax.bitcast_convert_type(xp, jnp.uint32)
    a = lax.bitcast_convert_type(p << jnp.uint32(16), F32)
    b = lax.bitcast_convert_type(p & jnp.uint32(0xFFFF0000), F32)
    return jnp.concatenate([a, b], axis=1)


def _tc_qkv(h, g, bln, wq, bq, wk, bk, wv, bv):
    def body(h_ref, g_ref, b_ref, wq_ref, bq_ref, wk_ref, bk_ref,
             wv_ref, bv_ref, q_ref, k_ref, v_ref):
        hn = _ln_block(h_ref[...], g_ref[...], b_ref[...])
        q_ref[...] = _pack_bf16(_dot(hn, wq_ref[...]) + bq_ref[...], NB)
        k_ref[...] = _pack_bf16(_dot(hn, wk_ref[...]) + bk_ref[...], NB)
        v_ref[...] = _dot(hn, wv_ref[...]) + bv_ref[...]

    wspec = pl.BlockSpec((HID, HID), lambda i: (0, 0))
    bspec = pl.BlockSpec((1, HID), lambda i: (0, 0))
    nspec = pl.BlockSpec((NB, HID), lambda i: (i, 0))
    pspec = pl.BlockSpec((NB, 128), lambda i: (i, 0))
    sds = jax.ShapeDtypeStruct((N, HID), F32)
    pds = jax.ShapeDtypeStruct((N, 128), F32)
    return pl.pallas_call(
        body,
        grid=(N // NB,),
        in_specs=[nspec, bspec, bspec, wspec, bspec, wspec, bspec, wspec, bspec],
        out_specs=[pspec, pspec, nspec],
        out_shape=[pds, pds, sds],
    )(h, g, bln, wq, bq, wk, bk, wv, bv)


def _tc_scores(qs, kt, eb_all, layer):
    def body(qs_ref, kt_ref, eb_ref, o_ref, ow_ref):
        d = lax.broadcasted_iota(jnp.int32, (HID, 16), 0)
        hh = lax.broadcasted_iota(jnp.int32, (HID, 16), 1)
        m = jnp.where(d // HD == hh, 1.0 / jnp.sqrt(jnp.float32(HD)), 0.0)
        qs = _unpack_bf16(qs_ref[...], EBK)
        kt = _unpack_bf16(kt_ref[...], EBK)
        s = _dot(qs * kt, m)
        eb = eb_ref[:, 16 * layer:16 * layer + 16]
        ex = jnp.exp(s + eb)
        o_ref[...] = ex
        ow_ref[...] = jnp.concatenate(
            [ex, jnp.zeros((EBK, 112), F32)], axis=1)

    return pl.pallas_call(
        body,
        grid=(E // EBK,),
        in_specs=[
            pl.BlockSpec((EBK, 128), lambda i: (i, 0)),
            pl.BlockSpec((EBK, 128), lambda i: (i, 0)),
            pl.BlockSpec((EBK, 64), lambda i: (i, 0)),
        ],
        out_specs=[pl.BlockSpec((EBK, 16), lambda i: (i, 0)),
                   pl.BlockSpec((EBK, 128), lambda i: (i, 0))],
        out_shape=[jax.ShapeDtypeStruct((E, 16), F32),
                   jax.ShapeDtypeStruct((E, 128), F32)],
    )(qs, kt, eb_all)


def _tc_recip(den):
    def body(d_ref, o_ref):
        s = d_ref[0, :, :16] + d_ref[1, :, :16]
        o_ref[...] = 1.0 / (s + 1e-16)

    return pl.pallas_call(
        body,
        grid=(NPAD // NB,),
        in_specs=[pl.BlockSpec((2, NB, 128), lambda i: (0, i, 0))],
        out_specs=pl.BlockSpec((NB, 16), lambda i: (i, 0)),
        out_shape=jax.ShapeDtypeStruct((NPAD, 16), F32),
    )(den)


def _tc_vr(v, recip):
    def body(v_ref, r_ref, o_ref):
        hh = lax.broadcasted_iota(jnp.int32, (16, HID), 0)
        d = lax.broadcasted_iota(jnp.int32, (16, HID), 1)
        s = jnp.where(d // HD == hh, 1.0, 0.0)
        o_ref[...] = _pack_bf16(v_ref[...] * _dot(r_ref[...], s), NB)

    return pl.pallas_call(
        body,
        grid=(N // NB,),
        in_specs=[
            pl.BlockSpec((NB, HID), lambda i: (i, 0)),
            pl.BlockSpec((NB, 16), lambda i: (i, 0)),
        ],
        out_specs=pl.BlockSpec((NB, 128), lambda i: (i, 0)),
        out_shape=jax.ShapeDtypeStruct((N, 128), F32),
    )(v, recip)


def _tc_msgs(ex, vrt):
    def body(ex_ref, vt_ref, o_ref):
        hh = lax.broadcasted_iota(jnp.int32, (16, HID), 0)
        d = lax.broadcasted_iota(jnp.int32, (16, HID), 1)
        s = jnp.where(d // HD == hh, 1.0, 0.0)
        m = _dot(ex_ref[...], s) * _unpack_bf16(vt_ref[...], EBK)
        o_ref[0] = m[:, :128]
        o_ref[1] = m[:, 128:]

    return pl.pallas_call(
        body,
        grid=(E // EBK,),
        in_specs=[
            pl.BlockSpec((EBK, 16), lambda i: (i, 0)),
            pl.BlockSpec((EBK, 128), lambda i: (i, 0)),
        ],
        out_specs=pl.BlockSpec((2, EBK, 128), lambda i: (0, i, 0)),
        out_shape=jax.ShapeDtypeStruct((2, E, 128), F32),
    )(ex, vrt)


def _tc_post(h, agg, wo, bo, g2, b2ln, w1, b1, w2, b2):
    def body(h_ref, a_ref, wo_ref, bo_ref, g_ref, bl_ref, w1_ref, b1_ref,
             w2_ref, b2_ref, o_ref):
        a = jnp.concatenate([a_ref[0], a_ref[1]], axis=1)
        h1 = h_ref[...] + _dot(a, wo_ref[...]) + bo_ref[...]
        hn2 = _ln_block(h1, g_ref[...], bl_ref[...])
        f = jnp.maximum(_dot(hn2, w1_ref[...]) + b1_ref[...], 0.0)
        o_ref[...] = h1 + _dot(f, w2_ref[...]) + b2_ref[...]

    bspec = pl.BlockSpec((1, HID), lambda i: (0, 0))
    return pl.pallas_call(
        body,
        grid=(N // NB,),
        in_specs=[
            pl.BlockSpec((NB, HID), lambda i: (i, 0)),
            pl.BlockSpec((2, NB, 128), lambda i: (0, i, 0)),
            pl.BlockSpec((HID, HID), lambda i: (0, 0)),
            bspec, bspec, bspec,
            pl.BlockSpec((HID, 4 * HID), lambda i: (0, 0)),
            pl.BlockSpec((1, 4 * HID), lambda i: (0, 0)),
            pl.BlockSpec((4 * HID, HID), lambda i: (0, 0)),
            bspec,
        ],
        out_specs=pl.BlockSpec((NB, HID), lambda i: (i, 0)),
        out_shape=jax.ShapeDtypeStruct((N, HID), F32),
    )(h, agg, wo, bo, g2, b2ln, w1, b1, w2, b2)


def _tc_out(h, w, b):
    def body(h_ref, w_ref, b_ref, o_ref):
        o_ref[...] = _dot(h_ref[...], w_ref[...]) + b_ref[...]

    return pl.pallas_call(
        body,
        grid=(N // NB,),
        in_specs=[
            pl.BlockSpec((NB, HID), lambda i: (i, 0)),
            pl.BlockSpec((HID, 128), lambda i: (0, 0)),
            pl.BlockSpec((1, 128), lambda i: (0, 0)),
        ],
        out_specs=pl.BlockSpec((NB, 128), lambda i: (i, 0)),
        out_shape=jax.ShapeDtypeStruct((N, 128), F32),
    )(h, w, b)


@functools.lru_cache(maxsize=1)
def _sc_mesh():
    return plsc.VectorSubcoreMesh(core_axis_name="c", subcore_axis_name="s")


def _stage_idx_flat(i_hbm, ibuf, start, n, cw, is_long):
    @pl.when(is_long)
    def _():
        pltpu.sync_copy(i_hbm.at[pl.ds(start * cw, (n + 1) * cw)], ibuf)

    @pl.when(jnp.logical_not(is_long))
    def _():
        pltpu.sync_copy(i_hbm.at[pl.ds(start * cw, n * cw)],
                        ibuf.at[pl.ds(0, n * cw)])


def _sc_gather_qk(q, k, src, tgt):
    n, r = NCHUNK // 16, NCHUNK % 16
    ngt = (n + 3) // 3

    @functools.partial(
        pl.kernel, mesh=_sc_mesh(),
        out_type=[jax.ShapeDtypeStruct((E, 128), F32),
                  jax.ShapeDtypeStruct((E, 128), F32)],
        scratch_types=[pltpu.VMEM(((n + 1) * CW,), jnp.int32),
                       pltpu.VMEM((CW, 128), F32), pltpu.VMEM((CW, 128), F32),
                       pltpu.VMEM((CW, 128), F32)]
                      + [pltpu.SemaphoreType.DMA] * 6,
    )
    def kfn(q_hbm, k_hbm, s_hbm, t_hbm, qs_hbm, kt_hbm,
            ibuf, b0, b1, b2, sg0, sg1, sg2, sw0, sw1, sw2):
        cid = lax.axis_index("c")
        sid = lax.axis_index("s")
        ncw = n + jnp.where(sid < r, 1, 0)
        start = sid * n + jnp.minimum(sid, r)

        bufs = (b0, b1, b2)
        sg, sw = (sg0, sg1, sg2), (sw0, sw1, sw2)

        def one_table(i_hbm, tbl_hbm, out_hbm):
            _stage_idx_flat(i_hbm, ibuf, start, n, CW, sid < r)

            def g_start(j, u):
                pltpu.make_async_copy(tbl_hbm.at[ibuf.at[pl.ds(j * CW, CW)]],
                                      bufs[u], sg[u]).start()

            def g_wait(u):
                pltpu.make_async_copy(tbl_hbm.at[ibuf.at[pl.ds(0, CW)]],
                                      bufs[u], sg[u]).wait()

            def wb_start(j, u):
                base = (start + j) * CW
                pltpu.make_async_copy(bufs[u], out_hbm.at[pl.ds(base, CW)],
                                      sw[u]).start()

            def wb_wait(u):
                pltpu.make_async_copy(bufs[u], out_hbm.at[pl.ds(0, CW)],
                                      sw[u]).wait()

            g_start(0, 0)
            g_start(1, 1)

            @pl.loop(0, ngt)
            def _(g):
                for u in (0, 1, 2):
                    j = g * 3 + u

                    @pl.when(j < ncw)
                    def _():
                        g_wait(u)
                        wb_start(j, u)

                        @pl.when(j + 2 < ncw)
                        def _():
                            @pl.when(j >= 1)
                            def _():
                                wb_wait((u + 2) % 3)

                            g_start(j + 2, (u + 2) % 3)

            wb_wait(0)
            wb_wait(1)
            wb_wait(2)

        @pl.when(cid == 0)
        def _():
            one_table(s_hbm, q_hbm, qs_hbm)

        @pl.when(cid == 1)
        def _():
            one_table(t_hbm, k_hbm, kt_hbm)

    return kfn(q, k, src, tgt)


def _sc_den(ex_wide, tgt, zeros128):
    @functools.partial(
        pl.kernel, mesh=_sc_mesh(),
        out_type=jax.ShapeDtypeStruct((2, NPAD, 128), F32),
        scratch_types=[pltpu.VMEM((CW,), jnp.int32),
                       pltpu.VMEM((CW,), jnp.int32),
                       pltpu.VMEM((CW, 128), F32),
                       pltpu.VMEM((CW, 128), F32),
                       pltpu.VMEM_SHARED((NPAD, 128), F32)]
                      + [pltpu.SemaphoreType.DMA] * 6,
    )
    def kfn(ex_hbm, t_hbm, z_hbm, den_hbm, ti0, ti1, exb0, exb1, dsh,
            sc0, sc1, si0, si1, ss0, ss1):
        cid = lax.axis_index("c")
        sid = lax.axis_index("s")
        w = sid * 2 + cid
        n, r = NCHUNK // 32, NCHUNK % 32
        ngp = (n + 2) // 2
        ncw = n + jnp.where(w < r, 1, 0)
        start = w * n + jnp.minimum(w, r)
        pltpu.sync_copy(z_hbm.at[pl.ds(sid * 640, 640)],
                        dsh.at[pl.ds(sid * 640, 640)])
        plsc.subcore_barrier()

        exb, sc = (exb0, exb1), (sc0, sc1)
        tib, si = (ti0, ti1), (si0, si1)
        ss = (ss0, ss1)

        def sc_start(u):
            pltpu.async_copy(exb[u], dsh.at[tib[u]], ss[u], add=True)

        def sc_wait(u):
            pltpu.make_async_copy(exb[u], dsh.at[tib[u]], ss[u]).wait()

        def cp_start(j, u):
            base = (start + j) * CW
            pltpu.make_async_copy(t_hbm.at[pl.ds(base, CW)], tib[u],
                                  si[u]).start()
            pltpu.make_async_copy(ex_hbm.at[pl.ds(base, CW)], exb[u],
                                  sc[u]).start()

        def cp_wait(u):
            pltpu.make_async_copy(t_hbm.at[pl.ds(0, CW)], tib[u],
                                  si[u]).wait()
            pltpu.make_async_copy(ex_hbm.at[pl.ds(0, CW)], exb[u],
                                  sc[u]).wait()

        cp_start(0, 0)

        @pl.loop(0, ngp)
        def _(g):
            for u in (0, 1):
                j = g * 2 + u

                @pl.when(j < ncw)
                def _():
                    cp_wait(u)

                    @pl.when(j + 1 < ncw)
                    def _():
                        @pl.when(j >= 1)
                        def _():
                            sc_wait(1 - u)

                        cp_start(j + 1, 1 - u)

                    sc_start(u)

        sc_wait(0)
        sc_wait(1)
        plsc.subcore_barrier()
        pltpu.sync_copy(dsh.at[pl.ds(sid * 640, 640)],
                        den_hbm.at[cid].at[pl.ds(sid * 640, 640)])

    return kfn(ex_wide, tgt, zeros128)


def _sc_gather_v(vr, tgt):
    n, r = NCHUNK // 32, NCHUNK % 32
    ngt = (n + 3) // 3

    @functools.partial(
        pl.kernel, mesh=_sc_mesh(),
        out_type=jax.ShapeDtypeStruct((E, 128), F32),
        scratch_types=[pltpu.VMEM(((n + 1) * CW,), jnp.int32),
                       pltpu.VMEM((CW, 128), F32), pltpu.VMEM((CW, 128), F32),
                       pltpu.VMEM((CW, 128), F32)]
                      + [pltpu.SemaphoreType.DMA] * 6,
    )
    def kfn(v_hbm, t_hbm, vt_hbm, tibuf, vb0, vb1, vb2,
            sg0, sg1, sg2, sw0, sw1, sw2):
        w = lax.axis_index("s") * 2 + lax.axis_index("c")
        ncw = n + jnp.where(w < r, 1, 0)
        start = w * n + jnp.minimum(w, r)
        _stage_idx_flat(t_hbm, tibuf, start, n, CW, w < r)

        vb, sg, sw = (vb0, vb1, vb2), (sg0, sg1, sg2), (sw0, sw1, sw2)

        def g_start(j, u):
            pltpu.make_async_copy(v_hbm.at[tibuf.at[pl.ds(j * CW, CW)]],
                                  vb[u], sg[u]).start()

        def g_wait(u):
            pltpu.make_async_copy(v_hbm.at[tibuf.at[pl.ds(0, CW)]],
                                  vb[u], sg[u]).wait()

        def wb_start(j, u):
            base = (start + j) * CW
            pltpu.make_async_copy(vb[u], vt_hbm.at[pl.ds(base, CW)],
                                  sw[u]).start()

        def wb_wait(u):
            pltpu.make_async_copy(vb[u], vt_hbm.at[pl.ds(0, CW)],
                                  sw[u]).wait()

        g_start(0, 0)
        g_start(1, 1)

        @pl.loop(0, ngt)
        def _(g):
            for u in (0, 1, 2):
                j = g * 3 + u

                @pl.when(j < ncw)
                def _():
                    g_wait(u)
                    wb_start(j, u)

                    @pl.when(j + 2 < ncw)
                    def _():
                        @pl.when(j >= 1)
                        def _():
                            wb_wait((u + 2) % 3)

                        g_start(j + 2, (u + 2) % 3)

        wb_wait(0)
        wb_wait(1)
        wb_wait(2)

    return kfn(vr, tgt)


def _sc_agg(msgs, src, zeros128):
    n, r = NCHUNK // 16, NCHUNK % 16
    ngp = (n + 2) // 2

    @functools.partial(
        pl.kernel, mesh=_sc_mesh(),
        out_type=jax.ShapeDtypeStruct((2, NPAD, 128), F32),
        scratch_types=[pltpu.VMEM((CW,), jnp.int32),
                       pltpu.VMEM((CW,), jnp.int32),
                       pltpu.VMEM((CW, 128), F32),
                       pltpu.VMEM((CW, 128), F32),
                       pltpu.VMEM_SHARED((NPAD, 128), F32)]
                      + [pltpu.SemaphoreType.DMA] * 6,
    )
    def kfn(m_hbm, s_hbm, z_hbm, agg_hbm, si0b, si1b, mb0, mb1, ash,
            sc0, sc1, si0, si1, ss0, ss1):
        cid = lax.axis_index("c")
        sid = lax.axis_index("s")
        ncw = n + jnp.where(sid < r, 1, 0)
        start = sid * n + jnp.minimum(sid, r)
        pltpu.sync_copy(z_hbm.at[pl.ds(sid * 640, 640)],
                        ash.at[pl.ds(sid * 640, 640)])
        plsc.subcore_barrier()

        mb, sc = (mb0, mb1), (sc0, sc1)
        sib, si = (si0b, si1b), (si0, si1)
        ss = (ss0, ss1)

        def sc_start(u):
            pltpu.async_copy(mb[u], ash.at[sib[u]], ss[u], add=True)

        def sc_wait(u):
            pltpu.make_async_copy(mb[u], ash.at[sib[u]], ss[u]).wait()

        def cp_start(j, u):
            base = (start + j) * CW
            pltpu.make_async_copy(s_hbm.at[pl.ds(base, CW)], sib[u],
                                  si[u]).start()
            pltpu.make_async_copy(m_hbm.at[cid].at[pl.ds(base, CW)], mb[u],
                                  sc[u]).start()

        def cp_wait(u):
            pltpu.make_async_copy(s_hbm.at[pl.ds(0, CW)], sib[u],
                                  si[u]).wait()
            pltpu.make_async_copy(m_hbm.at[cid].at[pl.ds(0, CW)], mb[u],
                                  sc[u]).wait()

        cp_start(0, 0)

        @pl.loop(0, ngp)
        def _(g):
            for u in (0, 1):
                j = g * 2 + u

                @pl.when(j < ncw)
                def _():
                    cp_wait(u)

                    @pl.when(j + 1 < ncw)
                    def _():
                        @pl.when(j >= 1)
                        def _():
                            sc_wait(1 - u)

                        cp_start(j + 1, 1 - u)

                    sc_start(u)

        sc_wait(0)
        sc_wait(1)
        plsc.subcore_barrier()
        pltpu.sync_copy(ash.at[pl.ds(sid * 640, 640)],
                        agg_hbm.at[cid].at[pl.ds(sid * 640, 640)])

    return kfn(msgs, src, zeros128)


def kernel(x, edge_index, curvature, params):
    p = params
    src = edge_index[0]
    tgt = edge_index[1]

    wb = jnp.concatenate(
        [jnp.pad(lp['wbias'], ((0, 0), (0, 8))) for lp in p['layers']], axis=1)
    bbp = jnp.concatenate(
        [jnp.concatenate([lp['bbias'], jnp.full((8,), -1e30, F32)])
         for lp in p['layers']])[None, :]
    c1w = p['c1w'].reshape(1, HID)
    c1b = p['c1b'][None, :]
    c2b = p['c2b'][None, :]
    out_w = jnp.pad(p['out_w'], ((0, 0), (0, 127)))
    out_b = jnp.pad(p['out_b'], (0, 127))[None, :]
    zeros128 = jnp.zeros((NPAD, 128), F32)

    eb_all = _tc_eb(curvature, c1w, c1b, p['c2w'], wb, c2b, bbp)
    h = _tc_in(x, p['in_w'], p['in_b'][None, :])

    for l, lp in enumerate(p['layers']):
        q, k, v = _tc_qkv(h, lp['ln1_g'][None, :], lp['ln1_b'][None, :],
                          lp['wq'], lp['bq'][None, :],
                          lp['wk'], lp['bk'][None, :],
                          lp['wv'], lp['bv'][None, :])
        qs, kt = _sc_gather_qk(q, k, src, tgt)
        ex, ex_wide = _tc_scores(qs, kt, eb_all, l)
        den = _sc_den(ex_wide, tgt, zeros128)
        recip = _tc_recip(den)
        vr = _tc_vr(v, recip)
        vrt = _sc_gather_v(vr, tgt)
        msgs = _tc_msgs(ex, vrt)
        agg = _sc_agg(msgs, src, zeros128)
        h = _tc_post(h, agg, lp['wo'], lp['bo'][None, :],
                     lp['ln2_g'][None, :], lp['ln2_b'][None, :],
                     lp['w1'], lp['b1'][None, :],
                     lp['w2'], lp['b2'][None, :])

    y = _tc_out(h, out_w, out_b)
    return y[:, :1]
